# Initial kernel scaffold; baseline (speedup 1.0000x reference)
import jax
import jax.numpy as jnp
from jax.experimental import pallas as pl


def kernel(x, edge_index, edge_weight, W_self, W_neigh, bias):
    raise NotImplementedError("write your pallas kernel here")


# speedup vs baseline: 3.1577x; 3.1577x over previous
import jax
import jax.numpy as jnp
from jax import lax
from jax.experimental import pallas as pl
from jax.experimental.pallas import tpu as pltpu
from jax.experimental.pallas import tpu_sc as plsc

N = 10000
NP = 10240
E = 320000
D = 128

NC = 2
NS = 16
NW = NC * NS
EPW = E // NW
CH = 80
NCHUNK = EPW // CH
RPT = NP // NS
ZR = 128
WR = NP // D


def _sc_body(x_hbm, src_hbm, dst_hbm, w_hbm, out0_hbm, out1_hbm, outw_hbm,
             srcv, dstv, wv, rows_v, wsum_loc, zbuf, agg_sh, sem):
    c = lax.axis_index("c")
    s = lax.axis_index("s")
    wid = s * NC + c

    def zrow(i, _):
        for j in range(D // 16):
            zbuf[i, pl.ds(16 * j, 16)] = jnp.zeros((16,), jnp.float32)
        return 0
    lax.fori_loop(0, ZR, zrow, 0)

    def zwrow(i, _):
        for j in range(D // 16):
            wsum_loc[i, pl.ds(16 * j, 16)] = jnp.zeros((16,), jnp.float32)
        return 0
    lax.fori_loop(0, WR, zwrow, 0)

    for k in range(RPT // ZR):
        pltpu.sync_copy(zbuf, agg_sh.at[pl.ds(s * RPT + k * ZR, ZR)])
    plsc.subcore_barrier()

    iota16 = lax.broadcasted_iota(jnp.int32, (16,), 0)

    def chunk(k, _):
        base = wid * EPW + k * CH
        pltpu.sync_copy(src_hbm.at[pl.ds(base, CH)], srcv)
        pltpu.sync_copy(dst_hbm.at[pl.ds(base, CH)], dstv)
        pltpu.sync_copy(w_hbm.at[pl.ds(base, CH)], wv)
        pltpu.async_copy(x_hbm.at[srcv], rows_v, sem).wait()

        def grp(g, _):
            w16 = wv[pl.ds(16 * g, 16)]
            d16 = dstv[pl.ds(16 * g, 16)]
            for r in range(16):
                w = w16[r]
                d = d16[r]
                i = 16 * g + r
                for j in range(D // 16):
                    rows_v[i, pl.ds(16 * j, 16)] = (
                        rows_v[i, pl.ds(16 * j, 16)] * w)
                row = lax.shift_right_logical(d, 7)
                colg = lax.shift_right_logical(d, 4) & 7
                onehot = jnp.where(iota16 == (d & 15), w, 0.0)
                plsc.addupdate(wsum_loc.at[row, pl.ds(colg * 16, 16)], onehot)
            return 0
        lax.fori_loop(0, CH // 16, grp, 0)

        pltpu.sync_copy(rows_v, agg_sh.at[dstv], add=True)
        return 0
    lax.fori_loop(0, NCHUNK, chunk, 0)

    plsc.subcore_barrier()

    @pl.when(c == 0)
    def _():
        pltpu.sync_copy(agg_sh.at[pl.ds(s * RPT, RPT)],
                        out0_hbm.at[pl.ds(s * RPT, RPT)])

    @pl.when(c == 1)
    def _():
        pltpu.sync_copy(agg_sh.at[pl.ds(s * RPT, RPT)],
                        out1_hbm.at[pl.ds(s * RPT, RPT)])

    pltpu.sync_copy(wsum_loc, outw_hbm.at[pl.ds(wid * WR, WR)])


@jax.jit
def _sc_aggregate(x, src, dst, w):
    mesh = plsc.VectorSubcoreMesh(core_axis_name="c", subcore_axis_name="s")
    f = pl.kernel(
        _sc_body,
        out_type=(jax.ShapeDtypeStruct((NP, D), jnp.float32),
                  jax.ShapeDtypeStruct((NP, D), jnp.float32),
                  jax.ShapeDtypeStruct((NW * WR, D), jnp.float32)),
        mesh=mesh,
        scratch_types=[
            pltpu.VMEM((CH,), jnp.int32),
            pltpu.VMEM((CH,), jnp.int32),
            pltpu.VMEM((CH,), jnp.float32),
            pltpu.VMEM((CH, D), jnp.float32),
            pltpu.VMEM((WR, D), jnp.float32),
            pltpu.VMEM((ZR, D), jnp.float32),
            pltpu.VMEM_SHARED((NP, D), jnp.float32),
            pltpu.SemaphoreType.DMA,
        ],
    )
    return f(x, src, dst, w)


def _tc_body(x_ref, p0_ref, p1_ref, w_ref, wst_ref, wnt_ref, b_ref, o_ref):
    agg = p0_ref[...] + p1_ref[...]
    wsum = jnp.sum(w_ref[...], axis=0)
    neigh = agg / jnp.maximum(wsum, 1e-8)
    out = (jnp.dot(x_ref[...], wst_ref[...], preferred_element_type=jnp.float32)
           + jnp.dot(neigh, wnt_ref[...], preferred_element_type=jnp.float32)
           + b_ref[...])
    n2 = jnp.sum(out * out, axis=-1, keepdims=True)
    o_ref[...] = out * lax.rsqrt(jnp.maximum(n2, 1e-24))


@jax.jit
def _tc_finish(x, p0, p1, wparts, wst, wnt, bias2d):
    R = 1024
    grid = (NP // R,)
    return pl.pallas_call(
        _tc_body,
        grid=grid,
        in_specs=[
            pl.BlockSpec((R, D), lambda i: (i, 0)),
            pl.BlockSpec((R, D), lambda i: (i, 0)),
            pl.BlockSpec((R, D), lambda i: (i, 0)),
            pl.BlockSpec((NW, R, 1), lambda i: (0, i, 0)),
            pl.BlockSpec((D, D), lambda i: (0, 0)),
            pl.BlockSpec((D, D), lambda i: (0, 0)),
            pl.BlockSpec((1, D), lambda i: (0, 0)),
        ],
        out_specs=pl.BlockSpec((R, D), lambda i: (i, 0)),
        out_shape=jax.ShapeDtypeStruct((N, D), jnp.float32),
    )(x, p0, p1, wparts, wst, wnt, bias2d)


def kernel(x, edge_index, edge_weight, W_self, W_neigh, bias):
    src = edge_index[0]
    dst = edge_index[1]
    p0, p1, wflat = _sc_aggregate(x, src, dst, edge_weight)
    wparts = wflat.reshape(NW, NP, 1)
    return _tc_finish(x, p0, p1, wparts, W_self.T, W_neigh.T,
                      bias.reshape(1, D))


# speedup vs baseline: 3.1891x; 1.0099x over previous
import jax
import jax.numpy as jnp
from jax import lax
from jax.experimental import pallas as pl
from jax.experimental.pallas import tpu as pltpu
from jax.experimental.pallas import tpu_sc as plsc

N = 10000
NP = 10240
E = 320000
D = 128

NC = 2
NS = 16
NW = NC * NS
EPW = E // NW
EPP = 10240
CH = 64
NCHUNK = EPP // CH
NB = 4
NI = 8
RPT = NP // NS
WR = NP // D


def _scale_chunk(buf, wsum_loc, ring, q, iota16):
    def grp(g, _):
        d16 = ring[q, 0, pl.ds(16 * g, 16)]
        w16 = lax.bitcast_convert_type(ring[q, 2, pl.ds(16 * g, 16)],
                                       jnp.float32)
        for r in range(16):
            w = w16[r]
            d = d16[r]
            i = 16 * g + r
            for j in range(D // 16):
                buf[i, pl.ds(16 * j, 16)] = buf[i, pl.ds(16 * j, 16)] * w
            row = lax.shift_right_logical(d, 7)
            colg = lax.shift_right_logical(d, 4) & 7
            onehot = jnp.where(iota16 == (d & 15), w, 0.0)
            plsc.addupdate(wsum_loc.at[row, pl.ds(colg * 16, 16)], onehot)
        return 0
    lax.fori_loop(0, CH // 16, grp, 0)


def _sc_body(x_hbm, epk_hbm, out0_hbm, out1_hbm, outw_hbm,
             ring, b0, b1, b2, b3, wsum_loc, agg_sh,
             gsem0, gsem1, gsem2, gsem3, ssem0, ssem1, ssem2, ssem3,
             isem0, isem1, isem2, isem3, isem4, isem5, isem6, isem7):
    c = lax.axis_index("c")
    s = lax.axis_index("s")
    wid = s * NC + c
    cbase = wid * NCHUNK

    bufs = (b0, b1, b2, b3)
    gsems = (gsem0, gsem1, gsem2, gsem3)
    ssems = (ssem0, ssem1, ssem2, ssem3)
    isems = (isem0, isem1, isem2, isem3, isem4, isem5, isem6, isem7)

    for q in range(4):
        pltpu.async_copy(epk_hbm.at[cbase + q], ring.at[q], isems[q])

    def zwrow(i, _):
        for j in range(D // 16):
            wsum_loc[i, pl.ds(16 * j, 16)] = jnp.zeros((16,), jnp.float32)
        return 0
    lax.fori_loop(0, WR, zwrow, 0)
    for k in range(RPT // WR):
        pltpu.sync_copy(wsum_loc, agg_sh.at[pl.ds(s * RPT + k * WR, WR)])

    def idx_of(q):
        return ring.at[q, 0, pl.ds(0, CH)]

    def src_of(q):
        return ring.at[q, 1, pl.ds(0, CH)]

    pltpu.make_async_copy(epk_hbm.at[cbase], ring.at[0], isems[0]).wait()
    pltpu.async_copy(x_hbm.at[src_of(0)], b0, gsems[0])
    pltpu.make_async_copy(epk_hbm.at[cbase + 1], ring.at[1], isems[1]).wait()
    pltpu.async_copy(x_hbm.at[src_of(1)], b1, gsems[1])

    plsc.subcore_barrier()

    iota16 = lax.broadcasted_iota(jnp.int32, (16,), 0)

    def step(t, _):
        for u in range(NI):
            k = NI * t + u
            b = u % NB
            q = u
            bp2 = (u + 2) % NB
            qp2 = (u + 2) % NI
            qp4 = (u + 4) % NI

            @pl.when(k >= 2)
            def _():
                pltpu.make_async_copy(
                    bufs[bp2], agg_sh.at[idx_of(qp2)], ssems[bp2]).wait()

            @pl.when(k + 4 < NCHUNK)
            def _():
                pltpu.async_copy(epk_hbm.at[cbase + k + 4], ring.at[qp4],
                                 isems[qp4])

            @pl.when(k + 2 < NCHUNK)
            def _():
                pltpu.make_async_copy(epk_hbm.at[cbase + k + 2],
                                      ring.at[qp2], isems[qp2]).wait()
                pltpu.async_copy(x_hbm.at[src_of(qp2)], bufs[bp2],
                                 gsems[bp2])

            pltpu.make_async_copy(x_hbm.at[src_of(q)], bufs[b],
                                  gsems[b]).wait()
            _scale_chunk(bufs[b], wsum_loc, ring, q, iota16)
            pltpu.async_copy(bufs[b], agg_sh.at[idx_of(q)], ssems[b],
                             add=True)
        return 0
    lax.fori_loop(0, NCHUNK // NI, step, 0)

    pltpu.make_async_copy(b2, agg_sh.at[idx_of(6)], ssems[2]).wait()
    pltpu.make_async_copy(b3, agg_sh.at[idx_of(7)], ssems[3]).wait()

    plsc.subcore_barrier()

    @pl.when(c == 0)
    def _():
        pltpu.sync_copy(agg_sh.at[pl.ds(s * RPT, RPT)],
                        out0_hbm.at[pl.ds(s * RPT, RPT)])

    @pl.when(c == 1)
    def _():
        pltpu.sync_copy(agg_sh.at[pl.ds(s * RPT, RPT)],
                        out1_hbm.at[pl.ds(s * RPT, RPT)])

    pltpu.sync_copy(wsum_loc, outw_hbm.at[pl.ds(wid * WR, WR)])


@jax.jit
def _sc_aggregate(x, epk):
    mesh = plsc.VectorSubcoreMesh(core_axis_name="c", subcore_axis_name="s")
    f = pl.kernel(
        _sc_body,
        out_type=(jax.ShapeDtypeStruct((NP, D), jnp.float32),
                  jax.ShapeDtypeStruct((NP, D), jnp.float32),
                  jax.ShapeDtypeStruct((NW * WR, D), jnp.float32)),
        mesh=mesh,
        scratch_types=[
            pltpu.VMEM((NI, 4, CH), jnp.int32),
            pltpu.VMEM((CH, D), jnp.float32),
            pltpu.VMEM((CH, D), jnp.float32),
            pltpu.VMEM((CH, D), jnp.float32),
            pltpu.VMEM((CH, D), jnp.float32),
            pltpu.VMEM((WR, D), jnp.float32),
            pltpu.VMEM_SHARED((NP, D), jnp.float32),
        ] + [pltpu.SemaphoreType.DMA] * 16,
    )
    return f(x, epk)


def _tc_body(x_ref, p0_ref, p1_ref, w_ref, wst_ref, wnt_ref, b_ref, o_ref):
    agg = p0_ref[...] + p1_ref[...]
    wsum = jnp.sum(w_ref[...], axis=0)
    neigh = agg / jnp.maximum(wsum, 1e-8)
    out = (jnp.dot(x_ref[...], wst_ref[...], preferred_element_type=jnp.float32)
           + jnp.dot(neigh, wnt_ref[...], preferred_element_type=jnp.float32)
           + b_ref[...])
    n2 = jnp.sum(out * out, axis=-1, keepdims=True)
    o_ref[...] = out * lax.rsqrt(jnp.maximum(n2, 1e-24))


@jax.jit
def _tc_finish(x, p0, p1, wparts, wst, wnt, bias2d):
    R = 1024
    grid = (NP // R,)
    return pl.pallas_call(
        _tc_body,
        grid=grid,
        in_specs=[
            pl.BlockSpec((R, D), lambda i: (i, 0)),
            pl.BlockSpec((R, D), lambda i: (i, 0)),
            pl.BlockSpec((R, D), lambda i: (i, 0)),
            pl.BlockSpec((NW, R, 1), lambda i: (0, i, 0)),
            pl.BlockSpec((D, D), lambda i: (0, 0)),
            pl.BlockSpec((D, D), lambda i: (0, 0)),
            pl.BlockSpec((1, D), lambda i: (0, 0)),
        ],
        out_specs=pl.BlockSpec((R, D), lambda i: (i, 0)),
        out_shape=jax.ShapeDtypeStruct((N, D), jnp.float32),
    )(x, p0, p1, wparts, wst, wnt, bias2d)


def kernel(x, edge_index, edge_weight, W_self, W_neigh, bias):
    pad = ((0, 0), (0, EPP - EPW))
    srcp = jnp.pad(edge_index[0].reshape(NW, EPW), pad)
    dstp = jnp.pad(edge_index[1].reshape(NW, EPW), pad)
    wb = jnp.pad(edge_weight.reshape(NW, EPW), pad).view(jnp.int32)
    epk = jnp.stack(
        [dstp.reshape(NW, NCHUNK, CH), srcp.reshape(NW, NCHUNK, CH),
         wb.reshape(NW, NCHUNK, CH),
         jnp.zeros((NW, NCHUNK, CH), jnp.int32)],
        axis=2).reshape(NW * NCHUNK, 4, CH)
    p0, p1, wflat = _sc_aggregate(x, epk)
    wparts = wflat.reshape(NW, NP, 1)
    return _tc_finish(x, p0, p1, wparts, W_self.T, W_neigh.T,
                      bias.reshape(1, D))


# speedup vs baseline: 3.2730x; 1.0263x over previous
import jax
import jax.numpy as jnp
from jax import lax
from jax.experimental import pallas as pl
from jax.experimental.pallas import tpu as pltpu
from jax.experimental.pallas import tpu_sc as plsc

N = 10000
NP = 10240
E = 320000
D = 128

NC = 2
NS = 16
NW = NC * NS
EPW = E // NW
EPP = 10240
CH = 64
NCHUNK = EPP // CH
SC2 = 128
NSUP = EPP // SC2
NB = 4
NI = 4
RPT = NP // NS
WR = NP // D


def _scale_chunk(buf, wsum_loc, ring, sj, h, iota16):
    def grp(g, _):
        d16 = ring[sj, 0, pl.ds(64 * h + 16 * g, 16)]
        w16 = lax.bitcast_convert_type(
            ring[sj, 2, pl.ds(64 * h + 16 * g, 16)], jnp.float32)
        for r in range(16):
            w = w16[r]
            d = d16[r]
            i = 16 * g + r
            for j in range(D // 16):
                buf[i, pl.ds(16 * j, 16)] = buf[i, pl.ds(16 * j, 16)] * w
            row = lax.shift_right_logical(d, 7)
            colg = lax.shift_right_logical(d, 4) & 7
            onehot = jnp.where(iota16 == (d & 15), w, 0.0)
            plsc.addupdate(wsum_loc.at[row, pl.ds(colg * 16, 16)], onehot)
        return 0
    lax.fori_loop(0, CH // 16, grp, 0)


def _sc_body(x_hbm, dst_hbm, src_hbm, wbt_hbm, out0_hbm, out1_hbm, outw_hbm,
             ring, b0, b1, b2, b3, wsum_loc, agg_sh,
             gsem0, gsem1, gsem2, gsem3, ssem0, ssem1, ssem2, ssem3,
             isem0, isem1, isem2, isem3, zsem):
    c = lax.axis_index("c")
    s = lax.axis_index("s")
    wid = s * NC + c
    jbase = wid * NSUP

    bufs = (b0, b1, b2, b3)
    gsems = (gsem0, gsem1, gsem2, gsem3)
    ssems = (ssem0, ssem1, ssem2, ssem3)
    isems = (isem0, isem1, isem2, isem3)

    def load_sup(j, slot):
        pltpu.async_copy(dst_hbm.at[jbase + j], ring.at[slot, 0], isems[slot])
        pltpu.async_copy(src_hbm.at[jbase + j], ring.at[slot, 1], isems[slot])
        pltpu.async_copy(wbt_hbm.at[jbase + j], ring.at[slot, 2], isems[slot])

    def wait_sup(j, slot):
        pltpu.make_async_copy(dst_hbm.at[jbase + j], ring.at[slot, 0],
                              isems[slot]).wait()
        pltpu.make_async_copy(src_hbm.at[jbase + j], ring.at[slot, 1],
                              isems[slot]).wait()
        pltpu.make_async_copy(wbt_hbm.at[jbase + j], ring.at[slot, 2],
                              isems[slot]).wait()

    for j in range(2):
        load_sup(j, j)

    def zwrow(i, _):
        for j in range(D // 16):
            wsum_loc[i, pl.ds(16 * j, 16)] = jnp.zeros((16,), jnp.float32)
        return 0
    lax.fori_loop(0, WR, zwrow, 0)
    for k in range(RPT // WR):
        pltpu.async_copy(wsum_loc, agg_sh.at[pl.ds(s * RPT + k * WR, WR)],
                         zsem)
    for k in range(RPT // WR):
        pltpu.make_async_copy(wsum_loc,
                              agg_sh.at[pl.ds(s * RPT + k * WR, WR)],
                              zsem).wait()

    def idx_of(k_slot, h):
        return ring.at[k_slot, 0, pl.ds(64 * h, CH)]

    def src_of(k_slot, h):
        return ring.at[k_slot, 1, pl.ds(64 * h, CH)]

    wait_sup(0, 0)
    pltpu.async_copy(x_hbm.at[src_of(0, 0)], b0, gsems[0])
    pltpu.async_copy(x_hbm.at[src_of(0, 1)], b1, gsems[1])

    plsc.subcore_barrier()

    iota16 = lax.broadcasted_iota(jnp.int32, (16,), 0)

    def step(t, _):
        for u in range(8):
            k = 8 * t + u
            b = u % NB
            sj = (u >> 1) % NI
            h = u & 1
            bp2 = (u + 2) % NB
            sj2 = ((u + 2) >> 1) % NI
            h2 = (u + 2) & 1

            @pl.when(k >= 2)
            def _():
                pltpu.make_async_copy(
                    bufs[bp2], agg_sh.at[idx_of(sj2, h2)], ssems[bp2]).wait()

            if h == 0:
                @pl.when((k >> 1) + 2 < NSUP)
                def _():
                    load_sup((k >> 1) + 2, ((u >> 1) + 2) % NI)

            @pl.when(k + 2 < NCHUNK)
            def _():
                if h2 == 0:
                    wait_sup((k >> 1) + 1, sj2)
                pltpu.async_copy(x_hbm.at[src_of(sj2, h2)], bufs[bp2],
                                 gsems[bp2])

            pltpu.make_async_copy(x_hbm.at[src_of(sj, h)], bufs[b],
                                  gsems[b]).wait()
            _scale_chunk(bufs[b], wsum_loc, ring, sj, h, iota16)
            pltpu.async_copy(bufs[b], agg_sh.at[idx_of(sj, h)], ssems[b],
                             add=True)
        return 0
    lax.fori_loop(0, NCHUNK // 8, step, 0)

    pltpu.make_async_copy(b2, agg_sh.at[idx_of(3, 0)], ssems[2]).wait()
    pltpu.make_async_copy(b3, agg_sh.at[idx_of(3, 1)], ssems[3]).wait()

    plsc.subcore_barrier()

    @pl.when(c == 0)
    def _():
        pltpu.sync_copy(agg_sh.at[pl.ds(s * RPT, RPT)],
                        out0_hbm.at[pl.ds(s * RPT, RPT)])

    @pl.when(c == 1)
    def _():
        pltpu.sync_copy(agg_sh.at[pl.ds(s * RPT, RPT)],
                        out1_hbm.at[pl.ds(s * RPT, RPT)])

    pltpu.sync_copy(wsum_loc, outw_hbm.at[pl.ds(wid * WR, WR)])


@jax.jit
def _sc_aggregate(x, dst2, src2, wbt2):
    mesh = plsc.VectorSubcoreMesh(core_axis_name="c", subcore_axis_name="s")
    f = pl.kernel(
        _sc_body,
        out_type=(jax.ShapeDtypeStruct((NP, D), jnp.float32),
                  jax.ShapeDtypeStruct((NP, D), jnp.float32),
                  jax.ShapeDtypeStruct((NW * WR, D), jnp.float32)),
        mesh=mesh,
        scratch_types=[
            pltpu.VMEM((NI, 3, SC2), jnp.int32),
            pltpu.VMEM((CH, D), jnp.float32),
            pltpu.VMEM((CH, D), jnp.float32),
            pltpu.VMEM((CH, D), jnp.float32),
            pltpu.VMEM((CH, D), jnp.float32),
            pltpu.VMEM((WR, D), jnp.float32),
            pltpu.VMEM_SHARED((NP, D), jnp.float32),
        ] + [pltpu.SemaphoreType.DMA] * 13,
    )
    return f(x, dst2, src2, wbt2)


def _tc_self_body(x_ref, wst_ref, b_ref, s_ref):
    s_ref[...] = (jnp.dot(x_ref[...], wst_ref[...],
                          preferred_element_type=jnp.float32) + b_ref[...])


@jax.jit
def _tc_self(x, wst, bias2d):
    R = 1024
    return pl.pallas_call(
        _tc_self_body,
        grid=(NP // R,),
        in_specs=[
            pl.BlockSpec((R, D), lambda i: (i, 0)),
            pl.BlockSpec((D, D), lambda i: (0, 0)),
            pl.BlockSpec((1, D), lambda i: (0, 0)),
        ],
        out_specs=pl.BlockSpec((R, D), lambda i: (i, 0)),
        out_shape=jax.ShapeDtypeStruct((N, D), jnp.float32),
    )(x, wst, bias2d)


def _tc_body(s_ref, p0_ref, p1_ref, w_ref, wnt_ref, o_ref):
    agg = p0_ref[...] + p1_ref[...]
    wsum = jnp.sum(w_ref[...], axis=0)
    neigh = agg / jnp.maximum(wsum, 1e-8)
    out = s_ref[...] + jnp.dot(neigh, wnt_ref[...],
                               preferred_element_type=jnp.float32)
    n2 = jnp.sum(out * out, axis=-1, keepdims=True)
    o_ref[...] = out * lax.rsqrt(jnp.maximum(n2, 1e-24))


@jax.jit
def _tc_finish(sself, p0, p1, wparts, wnt):
    R = 1024
    return pl.pallas_call(
        _tc_body,
        grid=(NP // R,),
        in_specs=[
            pl.BlockSpec((R, D), lambda i: (i, 0)),
            pl.BlockSpec((R, D), lambda i: (i, 0)),
            pl.BlockSpec((R, D), lambda i: (i, 0)),
            pl.BlockSpec((NW, R, 1), lambda i: (0, i, 0)),
            pl.BlockSpec((D, D), lambda i: (0, 0)),
        ],
        out_specs=pl.BlockSpec((R, D), lambda i: (i, 0)),
        out_shape=jax.ShapeDtypeStruct((N, D), jnp.float32),
    )(sself, p0, p1, wparts, wnt)


def kernel(x, edge_index, edge_weight, W_self, W_neigh, bias):
    pad = ((0, 0), (0, EPP - EPW))
    dst2 = jnp.pad(edge_index[1].reshape(NW, EPW), pad).reshape(NW * NSUP, SC2)
    src2 = jnp.pad(edge_index[0].reshape(NW, EPW), pad).reshape(NW * NSUP, SC2)
    wbt2 = jnp.pad(edge_weight.reshape(NW, EPW),
                   pad).view(jnp.int32).reshape(NW * NSUP, SC2)
    sself = _tc_self(x, W_self.T, bias.reshape(1, D))
    p0, p1, wflat = _sc_aggregate(x, dst2, src2, wbt2)
    wparts = wflat.reshape(NW, NP, 1)
    return _tc_finish(sself, p0, p1, wparts, W_neigh.T)


# speedup vs baseline: 3.2822x; 1.0028x over previous
import jax
import jax.numpy as jnp
from jax import lax
from jax.experimental import pallas as pl
from jax.experimental.pallas import tpu as pltpu
from jax.experimental.pallas import tpu_sc as plsc

N = 10000
NP = 10240
E = 320000
D = 128

NC = 2
NS = 16
NW = NC * NS
EPW = E // NW
EPP = 10240
CH = 64
NCHUNK = EPP // CH
SC2 = 128
NSUP = EPP // SC2
NB = 4
NI = 4
RPT = NP // NS
WR = NP // D


def _scale_chunk(buf, wsum_loc, ring, sj, h, iota16):
    def grp(g, _):
        d16 = ring[sj, 0, pl.ds(64 * h + 16 * g, 16)]
        w16 = lax.bitcast_convert_type(
            ring[sj, 2, pl.ds(64 * h + 16 * g, 16)], jnp.float32)
        for r in range(16):
            w = w16[r]
            d = d16[r]
            i = 16 * g + r
            for j in range(D // 16):
                buf[i, pl.ds(16 * j, 16)] = buf[i, pl.ds(16 * j, 16)] * w
            row = lax.shift_right_logical(d, 7)
            colg = lax.shift_right_logical(d, 4) & 7
            onehot = jnp.where(iota16 == (d & 15), w, 0.0)
            plsc.addupdate(wsum_loc.at[row, pl.ds(colg * 16, 16)], onehot)
        return 0
    lax.fori_loop(0, CH // 16, grp, 0)


def _sc_body(x_hbm, dst_hbm, src_hbm, wbt_hbm, out0_hbm, out1_hbm, outw_hbm,
             ring, b0, b1, b2, b3, wsum_loc, agg_sh,
             gsem0, gsem1, gsem2, gsem3, gsem4, gsem5, gsem6, gsem7,
             ssem0, ssem1, ssem2, ssem3,
             isem0, isem1, isem2, isem3, zsem):
    c = lax.axis_index("c")
    s = lax.axis_index("s")
    wid = s * NC + c
    jbase = wid * NSUP

    bufs = (b0, b1, b2, b3)
    gsems = ((gsem0, gsem1), (gsem2, gsem3), (gsem4, gsem5), (gsem6, gsem7))
    ssems = (ssem0, ssem1, ssem2, ssem3)
    isems = (isem0, isem1, isem2, isem3)

    def gather_issue(slot, h, b):
        pltpu.async_copy(x_hbm.at[ring.at[slot, 1, pl.ds(64 * h, 32)]],
                         bufs[b].at[pl.ds(0, 32)], gsems[b][0])
        pltpu.async_copy(x_hbm.at[ring.at[slot, 1, pl.ds(64 * h + 32, 32)]],
                         bufs[b].at[pl.ds(32, 32)], gsems[b][1])

    def gather_wait(slot, h, b):
        pltpu.make_async_copy(x_hbm.at[ring.at[slot, 1, pl.ds(64 * h, 32)]],
                              bufs[b].at[pl.ds(0, 32)], gsems[b][0]).wait()
        pltpu.make_async_copy(
            x_hbm.at[ring.at[slot, 1, pl.ds(64 * h + 32, 32)]],
            bufs[b].at[pl.ds(32, 32)], gsems[b][1]).wait()

    def load_sup(j, slot):
        pltpu.async_copy(dst_hbm.at[jbase + j], ring.at[slot, 0], isems[slot])
        pltpu.async_copy(src_hbm.at[jbase + j], ring.at[slot, 1], isems[slot])
        pltpu.async_copy(wbt_hbm.at[jbase + j], ring.at[slot, 2], isems[slot])

    def wait_sup(j, slot):
        pltpu.make_async_copy(dst_hbm.at[jbase + j], ring.at[slot, 0],
                              isems[slot]).wait()
        pltpu.make_async_copy(src_hbm.at[jbase + j], ring.at[slot, 1],
                              isems[slot]).wait()
        pltpu.make_async_copy(wbt_hbm.at[jbase + j], ring.at[slot, 2],
                              isems[slot]).wait()

    for j in range(2):
        load_sup(j, j)

    def zwrow(i, _):
        for j in range(D // 16):
            wsum_loc[i, pl.ds(16 * j, 16)] = jnp.zeros((16,), jnp.float32)
        return 0
    lax.fori_loop(0, WR, zwrow, 0)
    for k in range(RPT // WR):
        pltpu.async_copy(wsum_loc, agg_sh.at[pl.ds(s * RPT + k * WR, WR)],
                         zsem)
    for k in range(RPT // WR):
        pltpu.make_async_copy(wsum_loc,
                              agg_sh.at[pl.ds(s * RPT + k * WR, WR)],
                              zsem).wait()

    def idx_of(k_slot, h):
        return ring.at[k_slot, 0, pl.ds(64 * h, CH)]

    def src_of(k_slot, h):
        return ring.at[k_slot, 1, pl.ds(64 * h, CH)]

    wait_sup(0, 0)
    gather_issue(0, 0, 0)
    gather_issue(0, 1, 1)

    plsc.subcore_barrier()

    iota16 = lax.broadcasted_iota(jnp.int32, (16,), 0)

    def step(t, _):
        for u in range(8):
            k = 8 * t + u
            b = u % NB
            sj = (u >> 1) % NI
            h = u & 1
            bp2 = (u + 2) % NB
            sj2 = ((u + 2) >> 1) % NI
            h2 = (u + 2) & 1

            @pl.when(k >= 2)
            def _():
                pltpu.make_async_copy(
                    bufs[bp2], agg_sh.at[idx_of(sj2, h2)], ssems[bp2]).wait()

            if h == 0:
                @pl.when((k >> 1) + 2 < NSUP)
                def _():
                    load_sup((k >> 1) + 2, ((u >> 1) + 2) % NI)

            @pl.when(k + 2 < NCHUNK)
            def _():
                if h2 == 0:
                    wait_sup((k >> 1) + 1, sj2)
                gather_issue(sj2, h2, bp2)

            gather_wait(sj, h, b)
            _scale_chunk(bufs[b], wsum_loc, ring, sj, h, iota16)
            pltpu.async_copy(bufs[b], agg_sh.at[idx_of(sj, h)], ssems[b],
                             add=True)
        return 0
    lax.fori_loop(0, NCHUNK // 8, step, 0)

    pltpu.make_async_copy(b2, agg_sh.at[idx_of(3, 0)], ssems[2]).wait()
    pltpu.make_async_copy(b3, agg_sh.at[idx_of(3, 1)], ssems[3]).wait()

    plsc.subcore_barrier()

    @pl.when(c == 0)
    def _():
        pltpu.sync_copy(agg_sh.at[pl.ds(s * RPT, RPT)],
                        out0_hbm.at[pl.ds(s * RPT, RPT)])

    @pl.when(c == 1)
    def _():
        pltpu.sync_copy(agg_sh.at[pl.ds(s * RPT, RPT)],
                        out1_hbm.at[pl.ds(s * RPT, RPT)])

    pltpu.sync_copy(wsum_loc, outw_hbm.at[pl.ds(wid * WR, WR)])


@jax.jit
def _sc_aggregate(x, dst2, src2, wbt2):
    mesh = plsc.VectorSubcoreMesh(core_axis_name="c", subcore_axis_name="s")
    f = pl.kernel(
        _sc_body,
        out_type=(jax.ShapeDtypeStruct((NP, D), jnp.float32),
                  jax.ShapeDtypeStruct((NP, D), jnp.float32),
                  jax.ShapeDtypeStruct((NW * WR, D), jnp.float32)),
        mesh=mesh,
        scratch_types=[
            pltpu.VMEM((NI, 3, SC2), jnp.int32),
            pltpu.VMEM((CH, D), jnp.float32),
            pltpu.VMEM((CH, D), jnp.float32),
            pltpu.VMEM((CH, D), jnp.float32),
            pltpu.VMEM((CH, D), jnp.float32),
            pltpu.VMEM((WR, D), jnp.float32),
            pltpu.VMEM_SHARED((NP, D), jnp.float32),
        ] + [pltpu.SemaphoreType.DMA] * 17,
    )
    return f(x, dst2, src2, wbt2)


def _tc_self_body(x_ref, wst_ref, b_ref, s_ref):
    s_ref[...] = (jnp.dot(x_ref[...], wst_ref[...],
                          preferred_element_type=jnp.float32) + b_ref[...])


@jax.jit
def _tc_self(x, wst, bias2d):
    R = 1024
    return pl.pallas_call(
        _tc_self_body,
        grid=(NP // R,),
        in_specs=[
            pl.BlockSpec((R, D), lambda i: (i, 0)),
            pl.BlockSpec((D, D), lambda i: (0, 0)),
            pl.BlockSpec((1, D), lambda i: (0, 0)),
        ],
        out_specs=pl.BlockSpec((R, D), lambda i: (i, 0)),
        out_shape=jax.ShapeDtypeStruct((N, D), jnp.float32),
    )(x, wst, bias2d)


def _tc_body(s_ref, p0_ref, p1_ref, w_ref, wnt_ref, o_ref):
    agg = p0_ref[...] + p1_ref[...]
    wsum = jnp.sum(w_ref[...], axis=0)
    neigh = agg / jnp.maximum(wsum, 1e-8)
    out = s_ref[...] + jnp.dot(neigh, wnt_ref[...],
                               preferred_element_type=jnp.float32)
    n2 = jnp.sum(out * out, axis=-1, keepdims=True)
    o_ref[...] = out * lax.rsqrt(jnp.maximum(n2, 1e-24))


@jax.jit
def _tc_finish(sself, p0, p1, wparts, wnt):
    R = 1024
    return pl.pallas_call(
        _tc_body,
        grid=(NP // R,),
        in_specs=[
            pl.BlockSpec((R, D), lambda i: (i, 0)),
            pl.BlockSpec((R, D), lambda i: (i, 0)),
            pl.BlockSpec((R, D), lambda i: (i, 0)),
            pl.BlockSpec((NW, R, 1), lambda i: (0, i, 0)),
            pl.BlockSpec((D, D), lambda i: (0, 0)),
        ],
        out_specs=pl.BlockSpec((R, D), lambda i: (i, 0)),
        out_shape=jax.ShapeDtypeStruct((N, D), jnp.float32),
    )(sself, p0, p1, wparts, wnt)


def kernel(x, edge_index, edge_weight, W_self, W_neigh, bias):
    pad = ((0, 0), (0, EPP - EPW))
    dst2 = jnp.pad(edge_index[1].reshape(NW, EPW), pad).reshape(NW * NSUP, SC2)
    src2 = jnp.pad(edge_index[0].reshape(NW, EPW), pad).reshape(NW * NSUP, SC2)
    wbt2 = jnp.pad(edge_weight.reshape(NW, EPW),
                   pad).view(jnp.int32).reshape(NW * NSUP, SC2)
    sself = _tc_self(x, W_self.T, bias.reshape(1, D))
    p0, p1, wflat = _sc_aggregate(x, dst2, src2, wbt2)
    wparts = wflat.reshape(NW, NP, 1)
    return _tc_finish(sself, p0, p1, wparts, W_neigh.T)


# speedup vs baseline: 3.5375x; 1.0778x over previous
import jax
import jax.numpy as jnp
from jax import lax
from jax.experimental import pallas as pl
from jax.experimental.pallas import tpu as pltpu
from jax.experimental.pallas import tpu_sc as plsc

N = 10000
NP = 10240
E = 320000
D = 128

NC = 2
NS = 16
NW = NC * NS
EPW = E // NW
EPP = 10240
CH = 64
NCHUNK = EPP // CH
SC2 = 128
NSUP = EPP // SC2
NB = 4
NI = 4
RPT = NP // NS
WR = NP // D


def _scale_chunk(buf, wsum_loc, ring, sj, h, iota16):
    def grp(g, _):
        d16 = ring[sj, 0, pl.ds(64 * h + 16 * g, 16)]
        w16 = lax.bitcast_convert_type(
            ring[sj, 2, pl.ds(64 * h + 16 * g, 16)], jnp.float32)
        for r in range(16):
            w = w16[r]
            d = d16[r]
            i = 16 * g + r
            for j in range(D // 16):
                buf[i, pl.ds(16 * j, 16)] = buf[i, pl.ds(16 * j, 16)] * w
            row = lax.shift_right_logical(d, 7)
            colg = lax.shift_right_logical(d, 4) & 7
            onehot = jnp.where(iota16 == (d & 15), w, 0.0)
            plsc.addupdate(wsum_loc.at[row, pl.ds(colg * 16, 16)], onehot)
        return 0
    lax.fori_loop(0, CH // 16, grp, 0)


def _sc_body(x_hbm, ei_hbm, wbt_hbm, out0_hbm, out1_hbm, outw_hbm,
             ring, b0, b1, b2, b3, wsum_loc, agg_sh,
             gsem0, gsem1, gsem2, gsem3, gsem4, gsem5, gsem6, gsem7,
             ssem0, ssem1, ssem2, ssem3,
             isem0, isem1, isem2, isem3, zsem):
    c = lax.axis_index("c")
    s = lax.axis_index("s")
    wid = s * NC + c
    jbase = wid * NSUP

    bufs = (b0, b1, b2, b3)
    gsems = ((gsem0, gsem1), (gsem2, gsem3), (gsem4, gsem5), (gsem6, gsem7))
    ssems = (ssem0, ssem1, ssem2, ssem3)
    isems = (isem0, isem1, isem2, isem3)

    def gather_issue(slot, h, b):
        pltpu.async_copy(x_hbm.at[ring.at[slot, 1, pl.ds(64 * h, 32)]],
                         bufs[b].at[pl.ds(0, 32)], gsems[b][0])
        pltpu.async_copy(x_hbm.at[ring.at[slot, 1, pl.ds(64 * h + 32, 32)]],
                         bufs[b].at[pl.ds(32, 32)], gsems[b][1])

    def gather_wait(slot, h, b):
        pltpu.make_async_copy(x_hbm.at[ring.at[slot, 1, pl.ds(64 * h, 32)]],
                              bufs[b].at[pl.ds(0, 32)], gsems[b][0]).wait()
        pltpu.make_async_copy(
            x_hbm.at[ring.at[slot, 1, pl.ds(64 * h + 32, 32)]],
            bufs[b].at[pl.ds(32, 32)], gsems[b][1]).wait()

    def load_sup(j, slot):
        pltpu.async_copy(ei_hbm.at[1, jbase + j], ring.at[slot, 0],
                         isems[slot])
        pltpu.async_copy(ei_hbm.at[0, jbase + j], ring.at[slot, 1],
                         isems[slot])
        pltpu.async_copy(wbt_hbm.at[jbase + j], ring.at[slot, 2], isems[slot])

    def wait_sup(j, slot):
        pltpu.make_async_copy(ei_hbm.at[1, jbase + j], ring.at[slot, 0],
                              isems[slot]).wait()
        pltpu.make_async_copy(ei_hbm.at[0, jbase + j], ring.at[slot, 1],
                              isems[slot]).wait()
        pltpu.make_async_copy(wbt_hbm.at[jbase + j], ring.at[slot, 2],
                              isems[slot]).wait()

    for j in range(2):
        load_sup(j, j)

    def zwrow(i, _):
        for j in range(D // 16):
            wsum_loc[i, pl.ds(16 * j, 16)] = jnp.zeros((16,), jnp.float32)
        return 0
    lax.fori_loop(0, WR, zwrow, 0)
    for k in range(RPT // WR):
        pltpu.async_copy(wsum_loc, agg_sh.at[pl.ds(s * RPT + k * WR, WR)],
                         zsem)
    for k in range(RPT // WR):
        pltpu.make_async_copy(wsum_loc,
                              agg_sh.at[pl.ds(s * RPT + k * WR, WR)],
                              zsem).wait()

    def idx_of(k_slot, h):
        return ring.at[k_slot, 0, pl.ds(64 * h, CH)]

    def src_of(k_slot, h):
        return ring.at[k_slot, 1, pl.ds(64 * h, CH)]

    wait_sup(0, 0)
    gather_issue(0, 0, 0)
    gather_issue(0, 1, 1)

    plsc.subcore_barrier()

    iota16 = lax.broadcasted_iota(jnp.int32, (16,), 0)

    def step(t, _):
        for u in range(8):
            k = 8 * t + u
            b = u % NB
            sj = (u >> 1) % NI
            h = u & 1
            bp2 = (u + 2) % NB
            sj2 = ((u + 2) >> 1) % NI
            h2 = (u + 2) & 1

            @pl.when(k >= 2)
            def _():
                pltpu.make_async_copy(
                    bufs[bp2], agg_sh.at[idx_of(sj2, h2)], ssems[bp2]).wait()

            if h == 0:
                @pl.when((k >> 1) + 2 < NSUP)
                def _():
                    load_sup((k >> 1) + 2, ((u >> 1) + 2) % NI)

            @pl.when(k + 2 < NCHUNK)
            def _():
                if h2 == 0:
                    wait_sup((k >> 1) + 1, sj2)
                gather_issue(sj2, h2, bp2)

            gather_wait(sj, h, b)
            _scale_chunk(bufs[b], wsum_loc, ring, sj, h, iota16)
            pltpu.async_copy(bufs[b], agg_sh.at[idx_of(sj, h)], ssems[b],
                             add=True)
        return 0
    lax.fori_loop(0, NCHUNK // 8, step, 0)

    pltpu.make_async_copy(b2, agg_sh.at[idx_of(3, 0)], ssems[2]).wait()
    pltpu.make_async_copy(b3, agg_sh.at[idx_of(3, 1)], ssems[3]).wait()

    plsc.subcore_barrier()

    @pl.when(c == 0)
    def _():
        pltpu.sync_copy(agg_sh.at[pl.ds(s * RPT, RPT)],
                        out0_hbm.at[pl.ds(s * RPT, RPT)])

    @pl.when(c == 1)
    def _():
        pltpu.sync_copy(agg_sh.at[pl.ds(s * RPT, RPT)],
                        out1_hbm.at[pl.ds(s * RPT, RPT)])

    pltpu.sync_copy(wsum_loc, outw_hbm.at[pl.ds(wid * WR, WR)])


@jax.jit
def _sc_aggregate(x, ei3, wbt2):
    mesh = plsc.VectorSubcoreMesh(core_axis_name="c", subcore_axis_name="s")
    f = pl.kernel(
        _sc_body,
        out_type=(jax.ShapeDtypeStruct((NP, D), jnp.float32),
                  jax.ShapeDtypeStruct((NP, D), jnp.float32),
                  jax.ShapeDtypeStruct((NW * WR, D), jnp.float32)),
        mesh=mesh,
        scratch_types=[
            pltpu.VMEM((NI, 3, SC2), jnp.int32),
            pltpu.VMEM((CH, D), jnp.float32),
            pltpu.VMEM((CH, D), jnp.float32),
            pltpu.VMEM((CH, D), jnp.float32),
            pltpu.VMEM((CH, D), jnp.float32),
            pltpu.VMEM((WR, D), jnp.float32),
            pltpu.VMEM_SHARED((NP, D), jnp.float32),
        ] + [pltpu.SemaphoreType.DMA] * 17,
    )
    return f(x, ei3, wbt2)


def _tc_self_body(x_ref, ws_ref, b_ref, s_ref):
    s_ref[...] = lax.dot_general(
        x_ref[...], ws_ref[...], (((1,), (1,)), ((), ())),
        preferred_element_type=jnp.float32) + b_ref[...]


@jax.jit
def _tc_self(x, wst, bias2d):
    R = 1024
    return pl.pallas_call(
        _tc_self_body,
        grid=(NP // R,),
        in_specs=[
            pl.BlockSpec((R, D), lambda i: (i, 0)),
            pl.BlockSpec((D, D), lambda i: (0, 0)),
            pl.BlockSpec((1, D), lambda i: (0, 0)),
        ],
        out_specs=pl.BlockSpec((R, D), lambda i: (i, 0)),
        out_shape=jax.ShapeDtypeStruct((N, D), jnp.float32),
    )(x, wst, bias2d)


def _tc_body(s_ref, p0_ref, p1_ref, w_ref, wn_ref, o_ref):
    agg = p0_ref[...] + p1_ref[...]
    wsum = jnp.sum(w_ref[...], axis=0)
    neigh = agg / jnp.maximum(wsum, 1e-8)
    out = s_ref[...] + lax.dot_general(
        neigh, wn_ref[...], (((1,), (1,)), ((), ())),
        preferred_element_type=jnp.float32)
    n2 = jnp.sum(out * out, axis=-1, keepdims=True)
    o_ref[...] = out * lax.rsqrt(jnp.maximum(n2, 1e-24))


@jax.jit
def _tc_finish(sself, p0, p1, wparts, wn):
    R = 1024
    return pl.pallas_call(
        _tc_body,
        grid=(NP // R,),
        in_specs=[
            pl.BlockSpec((R, D), lambda i: (i, 0)),
            pl.BlockSpec((R, D), lambda i: (i, 0)),
            pl.BlockSpec((R, D), lambda i: (i, 0)),
            pl.BlockSpec((NW, R, 1), lambda i: (0, i, 0)),
            pl.BlockSpec((D, D), lambda i: (0, 0)),
        ],
        out_specs=pl.BlockSpec((R, D), lambda i: (i, 0)),
        out_shape=jax.ShapeDtypeStruct((N, D), jnp.float32),
    )(sself, p0, p1, wparts, wn)


def kernel(x, edge_index, edge_weight, W_self, W_neigh, bias):
    epad = NW * EPP - E
    ei3 = jnp.pad(edge_index, ((0, 0), (0, epad))).reshape(2, NW * NSUP, SC2)
    wbt2 = jnp.pad(edge_weight,
                   (0, epad)).view(jnp.int32).reshape(NW * NSUP, SC2)
    sself = _tc_self(x, W_self, bias.reshape(1, D))
    p0, p1, wflat = _sc_aggregate(x, ei3, wbt2)
    wparts = wflat.reshape(NW, NP, 1)
    return _tc_finish(sself, p0, p1, wparts, W_neigh)


# speedup vs baseline: 5.2509x; 1.4843x over previous
import jax
import jax.numpy as jnp
from jax import lax
from jax.experimental import pallas as pl
from jax.experimental.pallas import tpu as pltpu
from jax.experimental.pallas import tpu_sc as plsc

N = 10000
NP = 10240
E = 320000
D = 128

NC = 2
NS = 16
NW = NC * NS
EPW = E // NW
EPP = 10240
CH = 64
NCHUNK = EPP // CH
SC2 = 128
NSUP = EPP // SC2
NB = 4
NI = 4
RPT = NP // NS
WR = NP // D


def _scale_chunk(buf, wsum_loc, ring, sj, h, iota16):
    def grp(g, _):
        d16 = ring[sj, 0, pl.ds(64 * h + 16 * g, 16)]
        w16 = lax.bitcast_convert_type(
            ring[sj, 2, pl.ds(64 * h + 16 * g, 16)], jnp.float32)
        for r in range(16):
            w = w16[r]
            d = d16[r]
            i = 16 * g + r
            for j in range(D // 16):
                buf[i, pl.ds(16 * j, 16)] = buf[i, pl.ds(16 * j, 16)] * w
            row = lax.shift_right_logical(d, 7)
            colg = lax.shift_right_logical(d, 4) & 7
            onehot = jnp.where(iota16 == (d & 15), w, 0.0)
            plsc.addupdate(wsum_loc.at[row, pl.ds(colg * 16, 16)], onehot)
        return 0
    lax.fori_loop(0, CH // 16, grp, 0)


def _sc_body(x_hbm, ei_hbm, wbt_hbm, out0_hbm, out1_hbm, outw_hbm,
             ring, b0, b1, b2, b3, wsum_loc, agg_sh,
             gsem0, gsem1, gsem2, gsem3, gsem4, gsem5, gsem6, gsem7,
             ssem0, ssem1, ssem2, ssem3,
             isem0, isem1, isem2, isem3, zsem):
    c = lax.axis_index("c")
    s = lax.axis_index("s")
    wid = s * NC + c
    jbase = wid * NSUP

    bufs = (b0, b1, b2, b3)
    gsems = ((gsem0, gsem1), (gsem2, gsem3), (gsem4, gsem5), (gsem6, gsem7))
    ssems = (ssem0, ssem1, ssem2, ssem3)
    isems = (isem0, isem1, isem2, isem3)

    def gather_issue(slot, h, b):
        pltpu.async_copy(x_hbm.at[ring.at[slot, 1, pl.ds(64 * h, 32)]],
                         bufs[b].at[pl.ds(0, 32)], gsems[b][0])
        pltpu.async_copy(x_hbm.at[ring.at[slot, 1, pl.ds(64 * h + 32, 32)]],
                         bufs[b].at[pl.ds(32, 32)], gsems[b][1])

    def gather_wait(slot, h, b):
        pltpu.make_async_copy(x_hbm.at[ring.at[slot, 1, pl.ds(64 * h, 32)]],
                              bufs[b].at[pl.ds(0, 32)], gsems[b][0]).wait()
        pltpu.make_async_copy(
            x_hbm.at[ring.at[slot, 1, pl.ds(64 * h + 32, 32)]],
            bufs[b].at[pl.ds(32, 32)], gsems[b][1]).wait()

    def load_sup(j, slot):
        pltpu.async_copy(ei_hbm.at[1, jbase + j], ring.at[slot, 0],
                         isems[slot])
        pltpu.async_copy(ei_hbm.at[0, jbase + j], ring.at[slot, 1],
                         isems[slot])
        pltpu.async_copy(wbt_hbm.at[jbase + j], ring.at[slot, 2], isems[slot])

    def wait_sup(j, slot):
        pltpu.make_async_copy(ei_hbm.at[1, jbase + j], ring.at[slot, 0],
                              isems[slot]).wait()
        pltpu.make_async_copy(ei_hbm.at[0, jbase + j], ring.at[slot, 1],
                              isems[slot]).wait()
        pltpu.make_async_copy(wbt_hbm.at[jbase + j], ring.at[slot, 2],
                              isems[slot]).wait()

    for j in range(2):
        load_sup(j, j)

    def zwrow(i, _):
        for j in range(D // 16):
            wsum_loc[i, pl.ds(16 * j, 16)] = jnp.zeros((16,), jnp.float32)
        return 0
    lax.fori_loop(0, WR, zwrow, 0)
    for k in range(RPT // WR):
        pltpu.async_copy(wsum_loc, agg_sh.at[pl.ds(s * RPT + k * WR, WR)],
                         zsem)
    for k in range(RPT // WR):
        pltpu.make_async_copy(wsum_loc,
                              agg_sh.at[pl.ds(s * RPT + k * WR, WR)],
                              zsem).wait()

    def idx_of(k_slot, h):
        return ring.at[k_slot, 0, pl.ds(64 * h, CH)]

    def src_of(k_slot, h):
        return ring.at[k_slot, 1, pl.ds(64 * h, CH)]

    wait_sup(0, 0)
    gather_issue(0, 0, 0)
    gather_issue(0, 1, 1)

    plsc.subcore_barrier()

    iota16 = lax.broadcasted_iota(jnp.int32, (16,), 0)

    def step(t, _):
        for u in range(8):
            k = 8 * t + u
            b = u % NB
            sj = (u >> 1) % NI
            h = u & 1
            bp2 = (u + 2) % NB
            sj2 = ((u + 2) >> 1) % NI
            h2 = (u + 2) & 1

            @pl.when(k >= 2)
            def _():
                pltpu.make_async_copy(
                    bufs[bp2], agg_sh.at[idx_of(sj2, h2)], ssems[bp2]).wait()

            if h == 0:
                @pl.when((k >> 1) + 2 < NSUP)
                def _():
                    load_sup((k >> 1) + 2, ((u >> 1) + 2) % NI)

            @pl.when(k + 2 < NCHUNK)
            def _():
                if h2 == 0:
                    wait_sup((k >> 1) + 1, sj2)
                gather_issue(sj2, h2, bp2)

            gather_wait(sj, h, b)
            _scale_chunk(bufs[b], wsum_loc, ring, sj, h, iota16)
            pltpu.async_copy(bufs[b], agg_sh.at[idx_of(sj, h)], ssems[b],
                             add=True)
        return 0
    lax.fori_loop(0, NCHUNK // 8, step, 0)

    pltpu.make_async_copy(b2, agg_sh.at[idx_of(3, 0)], ssems[2]).wait()
    pltpu.make_async_copy(b3, agg_sh.at[idx_of(3, 1)], ssems[3]).wait()

    plsc.subcore_barrier()

    @pl.when(c == 0)
    def _():
        pltpu.sync_copy(agg_sh.at[pl.ds(s * RPT, RPT)],
                        out0_hbm.at[pl.ds(s * RPT, RPT)])

    @pl.when(c == 1)
    def _():
        pltpu.sync_copy(agg_sh.at[pl.ds(s * RPT, RPT)],
                        out1_hbm.at[pl.ds(s * RPT, RPT)])

    pltpu.sync_copy(wsum_loc, outw_hbm.at[pl.ds(wid * WR, WR)])


@jax.jit
def _sc_aggregate(x, ei3, wbt2):
    mesh = plsc.VectorSubcoreMesh(core_axis_name="c", subcore_axis_name="s")
    f = pl.kernel(
        _sc_body,
        out_type=(jax.ShapeDtypeStruct((NP, D), jnp.float32),
                  jax.ShapeDtypeStruct((NP, D), jnp.float32),
                  jax.ShapeDtypeStruct((NW * WR, D), jnp.float32)),
        mesh=mesh,
        scratch_types=[
            pltpu.VMEM((NI, 3, SC2), jnp.int32),
            pltpu.VMEM((CH, D), jnp.float32),
            pltpu.VMEM((CH, D), jnp.float32),
            pltpu.VMEM((CH, D), jnp.float32),
            pltpu.VMEM((CH, D), jnp.float32),
            pltpu.VMEM((WR, D), jnp.float32),
            pltpu.VMEM_SHARED((NP, D), jnp.float32),
        ] + [pltpu.SemaphoreType.DMA] * 17,
    )
    return f(x, ei3, wbt2)


def _tc_self_body(x_ref, ws_ref, b_ref, s_ref):
    s_ref[...] = lax.dot_general(
        x_ref[...], ws_ref[...], (((1,), (1,)), ((), ())),
        preferred_element_type=jnp.float32) + b_ref[...]


@jax.jit
def _tc_self(x, wst, bias2d):
    R = 1024
    return pl.pallas_call(
        _tc_self_body,
        grid=(NP // R,),
        in_specs=[
            pl.BlockSpec((R, D), lambda i: (i, 0)),
            pl.BlockSpec((D, D), lambda i: (0, 0)),
            pl.BlockSpec((1, D), lambda i: (0, 0)),
        ],
        out_specs=pl.BlockSpec((R, D), lambda i: (i, 0)),
        out_shape=jax.ShapeDtypeStruct((N, D), jnp.float32),
    )(x, wst, bias2d)


def _tc_body(s_ref, p0_ref, p1_ref, w_ref, wn_ref, o_ref):
    agg = p0_ref[...] + p1_ref[...]
    wsum = jnp.maximum(jnp.sum(w_ref[...], axis=0), 1e-8)
    rows = lax.broadcasted_iota(jnp.int32, (wsum.shape[0] * D, wsum.shape[0]),
                                0)
    cols8 = lax.broadcasted_iota(jnp.int32,
                                 (wsum.shape[0] * D, wsum.shape[0]), 1)
    e8 = jnp.where((rows >> 7) == cols8, 1.0, 0.0)
    t = lax.dot_general(e8, wsum, (((1,), (0,)), ((), ())),
                        preferred_element_type=jnp.float32)
    rid = lax.broadcasted_iota(jnp.int32, t.shape, 0)
    cid = lax.broadcasted_iota(jnp.int32, t.shape, 1)
    wcol = jnp.sum(jnp.where((rid & 127) == cid, t, 0.0), axis=-1,
                   keepdims=True)
    neigh = agg / wcol
    out = s_ref[...] + lax.dot_general(
        neigh, wn_ref[...], (((1,), (1,)), ((), ())),
        preferred_element_type=jnp.float32)
    n2 = jnp.sum(out * out, axis=-1, keepdims=True)
    o_ref[...] = out * lax.rsqrt(jnp.maximum(n2, 1e-24))


@jax.jit
def _tc_finish(sself, p0, p1, wparts, wn):
    R = 1024
    return pl.pallas_call(
        _tc_body,
        grid=(NP // R,),
        in_specs=[
            pl.BlockSpec((R, D), lambda i: (i, 0)),
            pl.BlockSpec((R, D), lambda i: (i, 0)),
            pl.BlockSpec((R, D), lambda i: (i, 0)),
            pl.BlockSpec((NW, R // D, D), lambda i: (0, i, 0)),
            pl.BlockSpec((D, D), lambda i: (0, 0)),
        ],
        out_specs=pl.BlockSpec((R, D), lambda i: (i, 0)),
        out_shape=jax.ShapeDtypeStruct((N, D), jnp.float32),
    )(sself, p0, p1, wparts, wn)


def kernel(x, edge_index, edge_weight, W_self, W_neigh, bias):
    epad = NW * EPP - E
    ei3 = jnp.pad(edge_index, ((0, 0), (0, epad))).reshape(2, NW * NSUP, SC2)
    wbt2 = jnp.pad(edge_weight,
                   (0, epad)).view(jnp.int32).reshape(NW * NSUP, SC2)
    sself = _tc_self(x, W_self, bias.reshape(1, D))
    p0, p1, wflat = _sc_aggregate(x, ei3, wbt2)
    wparts = wflat.reshape(NW, WR, D)
    return _tc_finish(sself, p0, p1, wparts, W_neigh)
